# baseline (speedup 1.0000x reference)
import functools

import jax
import jax.numpy as jnp
from jax import lax
from jax.experimental import pallas as pl
from jax.experimental.pallas import tpu as pltpu
from jax.experimental.pallas import tpu_sc as plsc

H = 128
LANES = 16
NG = 8
CHUNK = NG * LANES
EPS = 1e-12


def _rsqrt(v):
    i = lax.bitcast_convert_type(v, jnp.int32)
    i = jnp.int32(0x5F3759DF) - (i >> 1)
    y = lax.bitcast_convert_type(i, jnp.float32)
    half = jnp.float32(0.5) * v
    for _ in range(3):
        y = y * (jnp.float32(1.5) - half * y * y)
    return y


def _sc_body(tok_per_w, ids_hbm, pti_hbm, wword_hbm, pt_hbm, g_hbm, b_hbm,
             out_hbm, idx_v, pti_v, w_v, p_v, g_v, b_v, sem0, sem1):
    wid = lax.axis_index("s") * 2 + lax.axis_index("c")
    base = wid * tok_per_w

    pltpu.sync_copy(g_hbm, g_v)
    pltpu.sync_copy(b_hbm, b_v)

    lane = lax.iota(jnp.int32, LANES)
    rows = [lane + LANES * g for g in range(NG)]
    zero = jnp.zeros((LANES,), jnp.float32)

    def chunk_body(ch, _):
        tbase = base + ch * CHUNK
        pltpu.sync_copy(ids_hbm.at[pl.ds(tbase, CHUNK)], idx_v)
        pltpu.sync_copy(pti_hbm.at[pl.ds(tbase, CHUNK)], pti_v)
        cw = pltpu.async_copy(wword_hbm.at[idx_v], w_v, sem0)
        cp = pltpu.async_copy(pt_hbm.at[pti_v], p_v, sem1)
        cw.wait()
        cp.wait()

        def sum_body(h, carry):
            colv = jnp.full((LANES,), h, jnp.int32)
            out = []
            for g in range(NG):
                acc, acc2 = carry[2 * g], carry[2 * g + 1]
                x = (plsc.load_gather(w_v, [rows[g], colv])
                     + plsc.load_gather(p_v, [rows[g], colv]))
                plsc.store_scatter(w_v, [rows[g], colv], x)
                out.append(acc + x)
                out.append(acc2 + x * x)
            return tuple(out)

        sums = lax.fori_loop(0, H, sum_body, (zero,) * (2 * NG))

        rs = []
        cc = []
        for g in range(NG):
            mu = sums[2 * g] * jnp.float32(1.0 / H)
            var = sums[2 * g + 1] * jnp.float32(1.0 / H) - mu * mu
            r = _rsqrt(var + jnp.float32(EPS))
            rs.append(r)
            cc.append(mu * r)

        def norm_body(h, _):
            colv = jnp.full((LANES,), h, jnp.int32)
            gv = plsc.load_gather(g_v, [colv])
            bv = plsc.load_gather(b_v, [colv])
            for g in range(NG):
                x = plsc.load_gather(w_v, [rows[g], colv])
                y = (x * rs[g] - cc[g]) * gv + bv
                plsc.store_scatter(w_v, [rows[g], colv], y)
            return 0

        lax.fori_loop(0, H, norm_body, 0)
        pltpu.sync_copy(w_v, out_hbm.at[pl.ds(tbase, CHUNK)])
        return 0

    lax.fori_loop(0, tok_per_w // CHUNK, chunk_body, 0)


def kernel(input_ids, token_type_ids, position_ids, W_word, W_pos, W_type,
           gamma, beta):
    B, L = input_ids.shape
    P = W_pos.shape[0]
    N = B * L
    info = plsc.get_sparse_core_info()
    nw = info.num_cores * info.num_subcores
    tok_per_w = N // nw
    assert tok_per_w % CHUNK == 0

    ids_flat = input_ids.reshape(-1)
    pt_table = (W_type[:, None, :] + W_pos[None, :, :]).reshape(-1, H)
    pti_flat = (token_type_ids * P + position_ids).reshape(-1)

    mesh = plsc.VectorSubcoreMesh(core_axis_name="c", subcore_axis_name="s")
    run = pl.kernel(
        functools.partial(_sc_body, tok_per_w),
        out_type=jax.ShapeDtypeStruct((N, H), jnp.float32),
        mesh=mesh,
        compiler_params=pltpu.CompilerParams(needs_layout_passes=False),
        scratch_types=[
            pltpu.VMEM((CHUNK,), jnp.int32),
            pltpu.VMEM((CHUNK,), jnp.int32),
            pltpu.VMEM((CHUNK, H), jnp.float32),
            pltpu.VMEM((CHUNK, H), jnp.float32),
            pltpu.VMEM((H,), jnp.float32),
            pltpu.VMEM((H,), jnp.float32),
            pltpu.SemaphoreType.DMA,
            pltpu.SemaphoreType.DMA,
        ],
    )
    out = run(ids_flat, pti_flat, W_word, pt_table, gamma, beta)
    return out.reshape(B, L, H)


# speedup vs baseline: 1.0670x; 1.0670x over previous
import functools

import jax
import jax.numpy as jnp
from jax import lax
from jax.experimental import pallas as pl
from jax.experimental.pallas import tpu as pltpu
from jax.experimental.pallas import tpu_sc as plsc

H = 128
LANES = 16
NG = 8
CHUNK = NG * LANES
EPS = 1e-12


def _rsqrt(v):
    i = lax.bitcast_convert_type(v, jnp.int32)
    i = jnp.int32(0x5F3759DF) - (i >> 1)
    y = lax.bitcast_convert_type(i, jnp.float32)
    half = jnp.float32(0.5) * v
    for _ in range(3):
        y = y * (jnp.float32(1.5) - half * y * y)
    return y


def _sc_body(tok_per_w, ids_hbm, pti_hbm, wword_hbm, pt_hbm, g_hbm, b_hbm,
             out_hbm, i0, i1, t0, t1, w0, w1, p0, p1, o0, o1, g_v, b_v,
             si0, si1, st0, st1, sw0, sw1, sp0, sp1, so0, so1):
    ibuf = [i0, i1]
    tbuf = [t0, t1]
    wbuf = [w0, w1]
    pbuf = [p0, p1]
    obuf = [o0, o1]
    sem_i = [si0, si1]
    sem_t = [st0, st1]
    sem_w = [sw0, sw1]
    sem_p = [sp0, sp1]
    sem_o = [so0, so1]

    wid = lax.axis_index("s") * 2 + lax.axis_index("c")
    base = wid * tok_per_w
    nch = tok_per_w // CHUNK

    pltpu.sync_copy(g_hbm, g_v)
    pltpu.sync_copy(b_hbm, b_v)

    lane = lax.iota(jnp.int32, LANES)
    rows = [lane + LANES * g for g in range(NG)]
    zero = jnp.zeros((LANES,), jnp.float32)

    def idx_start(c, s):
        tb = base + c * CHUNK
        ci = pltpu.make_async_copy(ids_hbm.at[pl.ds(tb, CHUNK)], ibuf[s], sem_i[s])
        ct = pltpu.make_async_copy(pti_hbm.at[pl.ds(tb, CHUNK)], tbuf[s], sem_t[s])
        ci.start()
        ct.start()

    def idx_wait(s):
        pltpu.make_async_copy(ids_hbm.at[pl.ds(0, CHUNK)], ibuf[s], sem_i[s]).wait()
        pltpu.make_async_copy(pti_hbm.at[pl.ds(0, CHUNK)], tbuf[s], sem_t[s]).wait()

    def gather_start(s):
        pltpu.make_async_copy(wword_hbm.at[ibuf[s]], wbuf[s], sem_w[s]).start()
        pltpu.make_async_copy(pt_hbm.at[tbuf[s]], pbuf[s], sem_p[s]).start()

    def gather_wait(s):
        pltpu.make_async_copy(wword_hbm.at[ibuf[s]], wbuf[s], sem_w[s]).wait()
        pltpu.make_async_copy(pt_hbm.at[tbuf[s]], pbuf[s], sem_p[s]).wait()

    def out_start(c, s):
        tb = base + c * CHUNK
        pltpu.make_async_copy(obuf[s], out_hbm.at[pl.ds(tb, CHUNK)], sem_o[s]).start()

    def out_wait(s):
        pltpu.make_async_copy(obuf[s], out_hbm.at[pl.ds(0, CHUNK)], sem_o[s]).wait()

    def compute(s):
        w_v, p_v, o_v = wbuf[s], pbuf[s], obuf[s]

        def sum_body(h, carry):
            colv = jnp.full((LANES,), h, jnp.int32)
            out = []
            for g in range(NG):
                acc, acc2 = carry[2 * g], carry[2 * g + 1]
                x = (plsc.load_gather(w_v, [rows[g], colv])
                     + plsc.load_gather(p_v, [rows[g], colv]))
                plsc.store_scatter(w_v, [rows[g], colv], x)
                out.append(acc + x)
                out.append(acc2 + x * x)
            return tuple(out)

        sums = lax.fori_loop(0, H, sum_body, (zero,) * (2 * NG))

        rs = []
        cc = []
        for g in range(NG):
            mu = sums[2 * g] * jnp.float32(1.0 / H)
            var = sums[2 * g + 1] * jnp.float32(1.0 / H) - mu * mu
            r = _rsqrt(var + jnp.float32(EPS))
            rs.append(r)
            cc.append(mu * r)

        def norm_body(h, _):
            colv = jnp.full((LANES,), h, jnp.int32)
            gv = plsc.load_gather(g_v, [colv])
            bv = plsc.load_gather(b_v, [colv])
            for g in range(NG):
                x = plsc.load_gather(w_v, [rows[g], colv])
                y = (x * rs[g] - cc[g]) * gv + bv
                plsc.store_scatter(o_v, [rows[g], colv], y)
            return 0

        lax.fori_loop(0, H, norm_body, 0)

    idx_start(0, 0)
    idx_start(1, 1)
    idx_wait(0)
    gather_start(0)

    def pair_body(sp, _):
        c0 = 2 * sp

        gather_wait(0)

        @pl.when(c0 + 2 < nch)
        def _():
            idx_start(c0 + 2, 0)

        idx_wait(1)
        gather_start(1)

        @pl.when(c0 >= 2)
        def _():
            out_wait(0)

        compute(0)
        out_start(c0, 0)

        gather_wait(1)

        @pl.when(c0 + 3 < nch)
        def _():
            idx_start(c0 + 3, 1)

        @pl.when(c0 + 2 < nch)
        def _():
            idx_wait(0)
            gather_start(0)

        @pl.when(c0 >= 1)
        def _():
            out_wait(1)

        compute(1)
        out_start(c0 + 1, 1)
        return 0

    lax.fori_loop(0, nch // 2, pair_body, 0)
    out_wait(0)
    out_wait(1)


def kernel(input_ids, token_type_ids, position_ids, W_word, W_pos, W_type,
           gamma, beta):
    B, L = input_ids.shape
    P = W_pos.shape[0]
    N = B * L
    info = plsc.get_sparse_core_info()
    nw = info.num_cores * info.num_subcores
    tok_per_w = N // nw
    assert tok_per_w % (2 * CHUNK) == 0

    ids_flat = input_ids.reshape(-1)
    pt_table = (W_type[:, None, :] + W_pos[None, :, :]).reshape(-1, H)
    pti_flat = (token_type_ids * P + position_ids).reshape(-1)

    mesh = plsc.VectorSubcoreMesh(core_axis_name="c", subcore_axis_name="s")
    run = pl.kernel(
        functools.partial(_sc_body, tok_per_w),
        out_type=jax.ShapeDtypeStruct((N, H), jnp.float32),
        mesh=mesh,
        compiler_params=pltpu.CompilerParams(needs_layout_passes=False),
        scratch_types=(
            [pltpu.VMEM((CHUNK,), jnp.int32)] * 4
            + [pltpu.VMEM((CHUNK, H), jnp.float32)] * 6
            + [pltpu.VMEM((H,), jnp.float32)] * 2
            + [pltpu.SemaphoreType.DMA] * 10
        ),
    )
    out = run(ids_flat, pti_flat, W_word, pt_table, gamma, beta)
    return out.reshape(B, L, H)


# speedup vs baseline: 12.2611x; 11.4907x over previous
import functools

import jax
import jax.numpy as jnp
from jax import lax
from jax.experimental import pallas as pl
from jax.experimental.pallas import tpu as pltpu
from jax.experimental.pallas import tpu_sc as plsc

H = 128
CHUNK = 128
NSLOT = 4
EPS = 1e-12
BB = 8


def _sc_gather_body(tok_per_w, ids_hbm, wword_hbm, out_hbm, *refs):
    ibuf = refs[0:NSLOT]
    wbuf = refs[NSLOT:2 * NSLOT]
    sem_i = refs[2 * NSLOT:3 * NSLOT]
    sem_w = refs[3 * NSLOT:4 * NSLOT]
    sem_o = refs[4 * NSLOT:5 * NSLOT]

    wid = lax.axis_index("s") * 2 + lax.axis_index("c")
    base = wid * tok_per_w
    nch = tok_per_w // CHUNK

    def idx_start(c, s):
        pltpu.make_async_copy(
            ids_hbm.at[pl.ds(base + c * CHUNK, CHUNK)], ibuf[s], sem_i[s]).start()

    def idx_wait(s):
        pltpu.make_async_copy(
            ids_hbm.at[pl.ds(0, CHUNK)], ibuf[s], sem_i[s]).wait()

    def gather_start(s):
        pltpu.make_async_copy(wword_hbm.at[ibuf[s]], wbuf[s], sem_w[s]).start()

    def gather_wait(s):
        pltpu.make_async_copy(wword_hbm.at[ibuf[s]], wbuf[s], sem_w[s]).wait()

    def out_start(c, s):
        pltpu.make_async_copy(
            wbuf[s], out_hbm.at[pl.ds(base + c * CHUNK, CHUNK)], sem_o[s]).start()

    def out_wait(s):
        pltpu.make_async_copy(
            wbuf[s], out_hbm.at[pl.ds(0, CHUNK)], sem_o[s]).wait()

    for s in range(NSLOT):
        idx_start(s, s)
    idx_wait(0)
    gather_start(0)
    idx_wait(1)
    gather_start(1)

    def quad_body(q, _):
        for j in range(NSLOT):
            k = NSLOT * q + j
            gather_wait(j)
            out_start(k, j)

            @pl.when(k + NSLOT < nch)
            def _():
                idx_start(k + NSLOT, j)

            s2 = (j + 2) % NSLOT

            @pl.when(jnp.logical_and(k + 2 < nch, k >= 2))
            def _():
                out_wait(s2)

            @pl.when(k + 2 < nch)
            def _():
                idx_wait(s2)
                gather_start(s2)
        return 0

    lax.fori_loop(0, nch // NSLOT, quad_body, 0)
    for s in range(NSLOT):
        out_wait(s)


def _tc_ln_body(wg_ref, tt_ref, wpos_ref, wtype_ref, g_ref, b_ref, o_ref):
    x = wg_ref[...] + wpos_ref[...][None, :, :]
    tt = tt_ref[...].astype(jnp.float32).reshape(BB, tt_ref.shape[-1], 1)
    x = x + wtype_ref[0][None, None, :] + tt * (wtype_ref[1] - wtype_ref[0])[None, None, :]
    mu = jnp.mean(x, axis=-1, keepdims=True)
    xc = x - mu
    var = jnp.mean(xc * xc, axis=-1, keepdims=True)
    y = xc * lax.rsqrt(var + jnp.float32(EPS))
    o_ref[...] = y * g_ref[0][None, None, :] + b_ref[0][None, None, :]


def kernel(input_ids, token_type_ids, position_ids, W_word, W_pos, W_type,
           gamma, beta):
    B, L = input_ids.shape
    N = B * L
    info = plsc.get_sparse_core_info()
    nw = info.num_cores * info.num_subcores
    tok_per_w = N // nw
    assert tok_per_w % (NSLOT * CHUNK) == 0

    ids_flat = input_ids.reshape(-1)

    mesh = plsc.VectorSubcoreMesh(core_axis_name="c", subcore_axis_name="s")
    gather_run = pl.kernel(
        functools.partial(_sc_gather_body, tok_per_w),
        out_type=jax.ShapeDtypeStruct((N, H), jnp.float32),
        mesh=mesh,
        compiler_params=pltpu.CompilerParams(needs_layout_passes=False),
        scratch_types=(
            [pltpu.VMEM((CHUNK,), jnp.int32)] * NSLOT
            + [pltpu.VMEM((CHUNK, H), jnp.float32)] * NSLOT
            + [pltpu.SemaphoreType.DMA] * (3 * NSLOT)
        ),
    )
    wg = gather_run(ids_flat, W_word).reshape(B, L, H)

    tt3 = token_type_ids.reshape(B, 1, L)
    out = pl.pallas_call(
        _tc_ln_body,
        grid=(B // BB,),
        in_specs=[
            pl.BlockSpec((BB, L, H), lambda i: (i, 0, 0)),
            pl.BlockSpec((BB, 1, L), lambda i: (i, 0, 0)),
            pl.BlockSpec((L, H), lambda i: (0, 0)),
            pl.BlockSpec((2, H), lambda i: (0, 0)),
            pl.BlockSpec((1, H), lambda i: (0, 0)),
            pl.BlockSpec((1, H), lambda i: (0, 0)),
        ],
        out_specs=pl.BlockSpec((BB, L, H), lambda i: (i, 0, 0)),
        out_shape=jax.ShapeDtypeStruct((B, L, H), jnp.float32),
    )(wg, tt3, W_pos[:L], W_type, gamma.reshape(1, H), beta.reshape(1, H))
    return out
